# baseline (speedup 1.0000x reference)
import functools

import numpy as np
import jax
import jax.numpy as jnp
from jax import lax
from jax.experimental import pallas as pl
from jax.experimental.pallas import tpu as pltpu

_A = 9
_H = 84
_W = 84
_WP = 88
_NPOS = _H * _W
_NPOSP = _H * _WP
_NANC = _NPOS * _A
_CH = 2
_ROWS_PER_CHUNK = _CH * _WP
_GRID = _H // _CH
_PRE_NMS = 6000
_POST_NMS = 300
_NMS_THRESH = 0.7
_MIN_SIZE = 16.0
_CLIP_MAX = float(_H * 16 - 1)
_NMS_PAD = 6144
_NB = _NMS_PAD // 128


def _np_anchors():
    base_size = 16
    ratios = np.array([0.5, 1.0, 2.0])
    scales = np.array([8.0, 16.0, 32.0])

    def whctrs(anchor):
        w = anchor[2] - anchor[0] + 1
        h = anchor[3] - anchor[1] + 1
        return w, h, anchor[0] + 0.5 * (w - 1), anchor[1] + 0.5 * (h - 1)

    def mkanchors(ws, hs, x_ctr, y_ctr):
        ws = ws[:, None]
        hs = hs[:, None]
        return np.hstack((x_ctr - 0.5 * (ws - 1), y_ctr - 0.5 * (hs - 1),
                          x_ctr + 0.5 * (ws - 1), y_ctr + 0.5 * (hs - 1)))

    base_anchor = np.array([1.0, 1.0, base_size, base_size]) - 1.0
    w, h, x_ctr, y_ctr = whctrs(base_anchor)
    size = w * h
    ws = np.round(np.sqrt(size / ratios))
    hs = np.round(ws * ratios)
    ratio_anchors = mkanchors(ws, hs, x_ctr, y_ctr)
    levels = []
    for i in range(ratio_anchors.shape[0]):
        w, h, x_ctr, y_ctr = whctrs(ratio_anchors[i, :])
        levels.append(mkanchors(w * scales, h * scales, x_ctr, y_ctr))
    base = np.vstack(levels)

    shift_x = np.arange(_W) * 16
    shift_y = np.arange(_H) * 16
    sx, sy = np.meshgrid(shift_x, shift_y)
    shifts = np.stack([sx.ravel(), sy.ravel(), sx.ravel(), sy.ravel()], axis=1)
    anc = (base[None, :, :] + shifts[:, None, :]).astype(np.float32)
    aw = anc[:, :, 2] - anc[:, :, 0] + np.float32(1.0)
    ah = anc[:, :, 3] - anc[:, :, 1] + np.float32(1.0)
    cx = anc[:, :, 0] + np.float32(0.5) * aw
    cy = anc[:, :, 1] + np.float32(0.5) * ah

    def widen(v, fill):
        vv = np.full((_H, _WP, _A), fill, np.float32)
        vv[:, :_W, :] = v.reshape(_H, _W, _A)
        return vv.reshape(_NPOSP, _A)

    return widen(aw, 1.0), widen(ah, 1.0), widen(cx, 0.0), widen(cy, 0.0)


_AW, _AH, _ACX, _ACY = _np_anchors()


def _conv_decode_body(x_ref, w1_ref, b1_ref, w2_ref, b2_ref,
                      aw_ref, ah_ref, cx_ref, cy_ref, out_ref):
    g = pl.program_id(0)

    zrow = jnp.zeros((1, 256), dtype=jnp.float32)
    acc = jnp.zeros((_ROWS_PER_CHUNK, 512), dtype=jnp.float32)
    for dy in (-1, 0, 1):
        v = x_ref[pl.ds((_CH * g + 1 + dy) * _WP, _ROWS_PER_CHUNK), :]
        for dx in (-1, 0, 1):
            s = (dy + 1) * 3 + (dx + 1)
            if dx == -1:
                xs = jnp.concatenate([zrow, v[:-1]], axis=0)
            elif dx == 1:
                xs = jnp.concatenate([v[1:], zrow], axis=0)
            else:
                xs = v
            acc = acc + jnp.dot(xs, w1_ref[s],
                                preferred_element_type=jnp.float32)
    rpn = jnp.maximum(acc + b1_ref[0][None, :], 0.0)

    combo = jnp.dot(rpn, w2_ref[...],
                    preferred_element_type=jnp.float32) + b2_ref[0][None, :]

    bg = combo[:, 0:9]
    fg = combo[:, 9:18]
    m = jnp.maximum(bg, fg)
    eb = jnp.exp(bg - m)
    ef = jnp.exp(fg - m)
    prob = ef / (eb + ef)

    dxv = combo[:, 18:27]
    dyv = combo[:, 27:36]
    dwv = combo[:, 36:45]
    dhv = combo[:, 45:54]

    aw = aw_ref[...]
    ah = ah_ref[...]
    cx = cx_ref[...]
    cy = cy_ref[...]

    pcx = dxv * aw + cx
    pcy = dyv * ah + cy
    pw = jnp.exp(dwv) * aw
    ph = jnp.exp(dhv) * ah
    x1 = jnp.clip(pcx - 0.5 * pw, 0.0, _CLIP_MAX)
    y1 = jnp.clip(pcy - 0.5 * ph, 0.0, _CLIP_MAX)
    x2 = jnp.clip(pcx + 0.5 * pw, 0.0, _CLIP_MAX)
    y2 = jnp.clip(pcy + 0.5 * ph, 0.0, _CLIP_MAX)

    valid = ((x2 - x1 + 1.0) >= _MIN_SIZE) & ((y2 - y1 + 1.0) >= _MIN_SIZE)
    score = jnp.where(valid, prob, -1e9)

    out_ref[...] = jnp.concatenate(
        [score, x1, y1, x2, y2, jnp.zeros((_ROWS_PER_CHUNK, 3), jnp.float32)],
        axis=1)


def _conv_decode(x_pad, w1, b1, w2, b2, aw, ah, cx, cy):
    full = lambda shape: pl.BlockSpec(shape, lambda g: (0,) * len(shape))
    anc_spec = pl.BlockSpec((_ROWS_PER_CHUNK, _A), lambda g: (g, 0))
    return pl.pallas_call(
        _conv_decode_body,
        grid=(_GRID,),
        in_specs=[
            full(x_pad.shape),
            full(w1.shape),
            full(b1.shape),
            full(w2.shape),
            full(b2.shape),
            anc_spec, anc_spec, anc_spec, anc_spec,
        ],
        out_specs=pl.BlockSpec((_ROWS_PER_CHUNK, 48), lambda g: (g, 0)),
        out_shape=jax.ShapeDtypeStruct((_NPOSP, 48), jnp.float32),
    )(x_pad, w1, b1, w2, b2, aw, ah, cx, cy)


def _col(v_row):
    return jnp.transpose(jnp.broadcast_to(v_row, (128, 128)))


def _nms_body(x1_ref, y1_ref, x2_ref, y2_ref, supp_ref, area_ref, alive_ref,
              mkk_ref):
    area_ref[...] = ((x2_ref[...] - x1_ref[...] + 1.0) *
                     (y2_ref[...] - y1_ref[...] + 1.0))
    supp_ref[...] = jnp.ones((_NB, 128), jnp.float32)

    lane_j = lax.broadcasted_iota(jnp.int32, (128, 128), 1)
    sub_i = lax.broadcasted_iota(jnp.int32, (128, 128), 0)
    tri = lane_j > sub_i
    lane_row = lax.broadcasted_iota(jnp.int32, (1, 128), 1)

    def k_body(carry):
        k, kept = carry
        x1k = _col(x1_ref[pl.ds(k, 1), :])
        y1k = _col(y1_ref[pl.ds(k, 1), :])
        x2k = _col(x2_ref[pl.ds(k, 1), :])
        y2k = _col(y2_ref[pl.ds(k, 1), :])
        ak = _col(area_ref[pl.ds(k, 1), :])

        def cross_body(p, suppc):
            x1p = x1_ref[pl.ds(p, 1), :]
            y1p = y1_ref[pl.ds(p, 1), :]
            x2p = x2_ref[pl.ds(p, 1), :]
            y2p = y2_ref[pl.ds(p, 1), :]
            ap = area_ref[pl.ds(p, 1), :]
            alive_p = alive_ref[pl.ds(p, 1), :]
            ww = jnp.maximum(0.0, jnp.minimum(x2k, x2p) -
                             jnp.maximum(x1k, x1p) + 1.0)
            hh = jnp.maximum(0.0, jnp.minimum(y2k, y2p) -
                             jnp.maximum(y1k, y1p) + 1.0)
            inter = ww * hh
            iou = inter / (ak + ap - inter)
            hit = jnp.where(iou > _NMS_THRESH, 1.0, 0.0) * alive_p
            return jnp.maximum(suppc, jnp.max(hit, axis=1, keepdims=True))

        suppc = lax.fori_loop(0, k, cross_body, jnp.zeros((128, 1), jnp.float32))

        x1r = x1_ref[pl.ds(k, 1), :]
        y1r = y1_ref[pl.ds(k, 1), :]
        x2r = x2_ref[pl.ds(k, 1), :]
        y2r = y2_ref[pl.ds(k, 1), :]
        ar = area_ref[pl.ds(k, 1), :]
        ww = jnp.maximum(0.0, jnp.minimum(x2k, x2r) - jnp.maximum(x1k, x1r) + 1.0)
        hh = jnp.maximum(0.0, jnp.minimum(y2k, y2r) - jnp.maximum(y1k, y1r) + 1.0)
        inter = ww * hh
        iou = inter / (ak + ar - inter)
        mkk_ref[...] = jnp.where((iou > _NMS_THRESH) & tri, 1.0, 0.0)

        sr = jnp.transpose(jnp.broadcast_to(suppc, (128, 128)))[0:1, :]

        def ser_body(i, sr):
            row_i = mkk_ref[pl.ds(i, 1), :]
            si = jnp.sum(jnp.where(lane_row == i, sr, 0.0))
            return jnp.where(si < 0.5, jnp.maximum(sr, row_i), sr)

        sr = lax.fori_loop(0, 128, ser_body, sr)
        supp_ref[pl.ds(k, 1), :] = sr
        alive_ref[pl.ds(k, 1), :] = 1.0 - sr

        vw = x2r - x1r + 1.0
        vh = y2r - y1r + 1.0
        validk = jnp.where((vw >= _MIN_SIZE) & (vh >= _MIN_SIZE), 1.0, 0.0)
        kept = kept + jnp.sum((1.0 - sr) * validk)
        return k + 1, kept

    def k_cond(carry):
        k, kept = carry
        return (k < _NB) & (kept < float(_POST_NMS))

    lax.while_loop(k_cond, k_body, (jnp.int32(0), jnp.float32(0.0)))


def _nms(bx1, by1, bx2, by2):
    full = pl.BlockSpec((_NB, 128), lambda: (0, 0))
    return pl.pallas_call(
        _nms_body,
        in_specs=[full, full, full, full],
        out_specs=full,
        out_shape=jax.ShapeDtypeStruct((_NB, 128), jnp.float32),
        scratch_shapes=[
            pltpu.VMEM((_NB, 128), jnp.float32),
            pltpu.VMEM((_NB, 128), jnp.float32),
            pltpu.VMEM((128, 128), jnp.float32),
        ],
    )(bx1, by1, bx2, by2)


def kernel(base_feat, img_info, gt_boxes, W1, b1, W_cls, b_cls, W_bbox, b_bbox):
    x = base_feat[0].transpose(1, 2, 0)
    x = jnp.pad(x, ((1, 1), (0, _WP - _W), (0, 0)))
    x_pad = x.reshape((_H + 2) * _WP, 256)
    w1 = W1.transpose(2, 3, 1, 0).reshape(9, 256, 512)
    wcls = W_cls[:, :, 0, 0]
    wbox = W_bbox[:, :, 0, 0]
    perm = jnp.array([a * 4 + c for c in range(4) for a in range(_A)])
    w2 = jnp.concatenate(
        [wcls, wbox[perm], jnp.zeros((10, 512), jnp.float32)], axis=0).T
    b2 = jnp.concatenate(
        [b_cls, b_bbox[perm], jnp.zeros((10,), jnp.float32)])[None, :]

    outp = _conv_decode(x_pad, w1, b1[None, :], w2, b2,
                        jnp.asarray(_AW), jnp.asarray(_AH),
                        jnp.asarray(_ACX), jnp.asarray(_ACY))
    out = outp.reshape(_H, _WP, 48)[:, :_W, :].reshape(_NPOS, 48)

    scores = out[:, 0:9].reshape(-1)
    x1 = out[:, 9:18].reshape(-1)
    y1 = out[:, 18:27].reshape(-1)
    x2 = out[:, 27:36].reshape(-1)
    y2 = out[:, 36:45].reshape(-1)

    top_scores, top_idx = lax.top_k(scores, _PRE_NMS)
    tx1 = x1[top_idx]
    ty1 = y1[top_idx]
    tx2 = x2[top_idx]
    ty2 = y2[top_idx]

    def to_blocks(v, fill):
        return jnp.concatenate(
            [v, jnp.full((_NMS_PAD - _PRE_NMS,), fill, jnp.float32)]
        ).reshape(_NB, 128)

    supp = _nms(to_blocks(tx1, 0.0), to_blocks(ty1, 0.0),
                to_blocks(tx2, -1.0), to_blocks(ty2, -1.0))
    suppressed = supp.reshape(-1)[:_PRE_NMS] > 0.5

    masked = jnp.where(suppressed, -1e9, top_scores)
    _, keep = lax.top_k(masked, _POST_NMS)
    final = jnp.stack(
        [jnp.zeros((_POST_NMS,), jnp.float32),
         tx1[keep], ty1[keep], tx2[keep], ty2[keep]], axis=1)
    return final, jnp.zeros(()), jnp.zeros(())


# speedup vs baseline: 1.1912x; 1.1912x over previous
import functools

import numpy as np
import jax
import jax.numpy as jnp
from jax import lax
from jax.experimental import pallas as pl
from jax.experimental.pallas import tpu as pltpu

_A = 9
_H = 84
_W = 84
_WP = 88
_NPOS = _H * _W
_NPOSP = _H * _WP
_NANC = _NPOS * _A
_CH = 2
_ROWS_PER_CHUNK = _CH * _WP
_GRID = _H // _CH
_PRE_NMS = 6000
_POST_NMS = 300
_NMS_THRESH = 0.7
_MIN_SIZE = 16.0
_CLIP_MAX = float(_H * 16 - 1)
_NMS_PAD = 6144
_NB = _NMS_PAD // 128


def _np_anchors():
    base_size = 16
    ratios = np.array([0.5, 1.0, 2.0])
    scales = np.array([8.0, 16.0, 32.0])

    def whctrs(anchor):
        w = anchor[2] - anchor[0] + 1
        h = anchor[3] - anchor[1] + 1
        return w, h, anchor[0] + 0.5 * (w - 1), anchor[1] + 0.5 * (h - 1)

    def mkanchors(ws, hs, x_ctr, y_ctr):
        ws = ws[:, None]
        hs = hs[:, None]
        return np.hstack((x_ctr - 0.5 * (ws - 1), y_ctr - 0.5 * (hs - 1),
                          x_ctr + 0.5 * (ws - 1), y_ctr + 0.5 * (hs - 1)))

    base_anchor = np.array([1.0, 1.0, base_size, base_size]) - 1.0
    w, h, x_ctr, y_ctr = whctrs(base_anchor)
    size = w * h
    ws = np.round(np.sqrt(size / ratios))
    hs = np.round(ws * ratios)
    ratio_anchors = mkanchors(ws, hs, x_ctr, y_ctr)
    levels = []
    for i in range(ratio_anchors.shape[0]):
        w, h, x_ctr, y_ctr = whctrs(ratio_anchors[i, :])
        levels.append(mkanchors(w * scales, h * scales, x_ctr, y_ctr))
    base = np.vstack(levels)

    shift_x = np.arange(_W) * 16
    shift_y = np.arange(_H) * 16
    sx, sy = np.meshgrid(shift_x, shift_y)
    shifts = np.stack([sx.ravel(), sy.ravel(), sx.ravel(), sy.ravel()], axis=1)
    anc = (base[None, :, :] + shifts[:, None, :]).astype(np.float32)
    aw = anc[:, :, 2] - anc[:, :, 0] + np.float32(1.0)
    ah = anc[:, :, 3] - anc[:, :, 1] + np.float32(1.0)
    cx = anc[:, :, 0] + np.float32(0.5) * aw
    cy = anc[:, :, 1] + np.float32(0.5) * ah

    def widen(v, fill):
        vv = np.full((_H, _WP, _A), fill, np.float32)
        vv[:, :_W, :] = v.reshape(_H, _W, _A)
        return vv.reshape(_NPOSP, _A)

    return widen(aw, 1.0), widen(ah, 1.0), widen(cx, 0.0), widen(cy, 0.0)


_AW, _AH, _ACX, _ACY = _np_anchors()


def _conv_decode_body(x_ref, w1_ref, b1_ref, w2_ref, b2_ref,
                      aw_ref, ah_ref, cx_ref, cy_ref, out_ref):
    g = pl.program_id(0)

    zrow = jnp.zeros((1, 256), dtype=jnp.float32)
    acc = jnp.zeros((_ROWS_PER_CHUNK, 512), dtype=jnp.float32)
    for dy in (-1, 0, 1):
        v = x_ref[pl.ds((_CH * g + 1 + dy) * _WP, _ROWS_PER_CHUNK), :]
        for dx in (-1, 0, 1):
            s = (dy + 1) * 3 + (dx + 1)
            if dx == -1:
                xs = jnp.concatenate([zrow, v[:-1]], axis=0)
            elif dx == 1:
                xs = jnp.concatenate([v[1:], zrow], axis=0)
            else:
                xs = v
            acc = acc + jnp.dot(xs, w1_ref[s],
                                preferred_element_type=jnp.float32)
    rpn = jnp.maximum(acc + b1_ref[0][None, :], 0.0)

    combo = jnp.dot(rpn, w2_ref[...],
                    preferred_element_type=jnp.float32) + b2_ref[0][None, :]

    bg = combo[:, 0:9]
    fg = combo[:, 9:18]
    m = jnp.maximum(bg, fg)
    eb = jnp.exp(bg - m)
    ef = jnp.exp(fg - m)
    prob = ef / (eb + ef)

    dxv = combo[:, 18:27]
    dyv = combo[:, 27:36]
    dwv = combo[:, 36:45]
    dhv = combo[:, 45:54]

    aw = aw_ref[...]
    ah = ah_ref[...]
    cx = cx_ref[...]
    cy = cy_ref[...]

    pcx = dxv * aw + cx
    pcy = dyv * ah + cy
    pw = jnp.exp(dwv) * aw
    ph = jnp.exp(dhv) * ah
    x1 = jnp.clip(pcx - 0.5 * pw, 0.0, _CLIP_MAX)
    y1 = jnp.clip(pcy - 0.5 * ph, 0.0, _CLIP_MAX)
    x2 = jnp.clip(pcx + 0.5 * pw, 0.0, _CLIP_MAX)
    y2 = jnp.clip(pcy + 0.5 * ph, 0.0, _CLIP_MAX)

    valid = ((x2 - x1 + 1.0) >= _MIN_SIZE) & ((y2 - y1 + 1.0) >= _MIN_SIZE)
    score = jnp.where(valid, prob, -1e9)

    out_ref[...] = jnp.concatenate(
        [score, x1, y1, x2, y2, jnp.zeros((_ROWS_PER_CHUNK, 3), jnp.float32)],
        axis=1)


def _conv_decode(x_pad, w1, b1, w2, b2, aw, ah, cx, cy):
    full = lambda shape: pl.BlockSpec(shape, lambda g: (0,) * len(shape))
    anc_spec = pl.BlockSpec((_ROWS_PER_CHUNK, _A), lambda g: (g, 0))
    return pl.pallas_call(
        _conv_decode_body,
        grid=(_GRID,),
        in_specs=[
            full(x_pad.shape),
            full(w1.shape),
            full(b1.shape),
            full(w2.shape),
            full(b2.shape),
            anc_spec, anc_spec, anc_spec, anc_spec,
        ],
        out_specs=pl.BlockSpec((_ROWS_PER_CHUNK, 48), lambda g: (g, 0)),
        out_shape=jax.ShapeDtypeStruct((_NPOSP, 48), jnp.float32),
    )(x_pad, w1, b1, w2, b2, aw, ah, cx, cy)


def _col(v_row):
    return jnp.transpose(jnp.broadcast_to(v_row, (128, 128)))


def _nms_body(x1_ref, y1_ref, x2_ref, y2_ref, supp_ref, area_ref, alive_ref):
    area_ref[...] = ((x2_ref[...] - x1_ref[...] + 1.0) *
                     (y2_ref[...] - y1_ref[...] + 1.0))
    supp_ref[...] = jnp.ones((_NB, 128), jnp.float32)

    lane_j = lax.broadcasted_iota(jnp.int32, (128, 128), 1)
    sub_i = lax.broadcasted_iota(jnp.int32, (128, 128), 0)
    tri = lane_j > sub_i

    def k_body(carry):
        k, kept = carry
        x1k = _col(x1_ref[pl.ds(k, 1), :])
        y1k = _col(y1_ref[pl.ds(k, 1), :])
        x2k = _col(x2_ref[pl.ds(k, 1), :])
        y2k = _col(y2_ref[pl.ds(k, 1), :])
        ak = _col(area_ref[pl.ds(k, 1), :])

        def cross_body(p, suppc):
            x1p = x1_ref[pl.ds(p, 1), :]
            y1p = y1_ref[pl.ds(p, 1), :]
            x2p = x2_ref[pl.ds(p, 1), :]
            y2p = y2_ref[pl.ds(p, 1), :]
            ap = area_ref[pl.ds(p, 1), :]
            alive_p = alive_ref[pl.ds(p, 1), :]
            ww = jnp.maximum(0.0, jnp.minimum(x2k, x2p) -
                             jnp.maximum(x1k, x1p) + 1.0)
            hh = jnp.maximum(0.0, jnp.minimum(y2k, y2p) -
                             jnp.maximum(y1k, y1p) + 1.0)
            inter = ww * hh
            iou = inter / (ak + ap - inter)
            hit = jnp.where(iou > _NMS_THRESH, 1.0, 0.0) * alive_p
            return jnp.maximum(suppc, jnp.max(hit, axis=1, keepdims=True))

        suppc = lax.fori_loop(0, k, cross_body, jnp.zeros((128, 1), jnp.float32))

        x1r = x1_ref[pl.ds(k, 1), :]
        y1r = y1_ref[pl.ds(k, 1), :]
        x2r = x2_ref[pl.ds(k, 1), :]
        y2r = y2_ref[pl.ds(k, 1), :]
        ar = area_ref[pl.ds(k, 1), :]
        ww = jnp.maximum(0.0, jnp.minimum(x2k, x2r) - jnp.maximum(x1k, x1r) + 1.0)
        hh = jnp.maximum(0.0, jnp.minimum(y2k, y2r) - jnp.maximum(y1k, y1r) + 1.0)
        inter = ww * hh
        iou = inter / (ak + ar - inter)
        mkk2 = jnp.where((iou > _NMS_THRESH) & (sub_i > lane_j), 1.0, 0.0)

        def fx_cond(c):
            t, delta, _ = c
            return (delta > 0.5) & (t < 130)

        def fx_body(c):
            t, _, scol = c
            sm = jnp.transpose(jnp.broadcast_to(scol, (128, 128)))
            contrib = jnp.max(mkk2 * (1.0 - sm), axis=1, keepdims=True)
            scol_new = jnp.maximum(suppc, contrib)
            delta = jnp.sum(jnp.abs(scol_new - scol))
            return t + 1, delta, scol_new

        _, _, scol = lax.while_loop(
            fx_cond, fx_body,
            (jnp.int32(0), jnp.float32(1.0), suppc))
        sr = jnp.transpose(jnp.broadcast_to(scol, (128, 128)))[0:1, :]
        supp_ref[pl.ds(k, 1), :] = sr
        alive_ref[pl.ds(k, 1), :] = 1.0 - sr

        vw = x2r - x1r + 1.0
        vh = y2r - y1r + 1.0
        validk = jnp.where((vw >= _MIN_SIZE) & (vh >= _MIN_SIZE), 1.0, 0.0)
        kept = kept + jnp.sum((1.0 - sr) * validk)
        return k + 1, kept

    def k_cond(carry):
        k, kept = carry
        return (k < _NB) & (kept < float(_POST_NMS))

    lax.while_loop(k_cond, k_body, (jnp.int32(0), jnp.float32(0.0)))


def _nms(bx1, by1, bx2, by2):
    full = pl.BlockSpec((_NB, 128), lambda: (0, 0))
    return pl.pallas_call(
        _nms_body,
        in_specs=[full, full, full, full],
        out_specs=full,
        out_shape=jax.ShapeDtypeStruct((_NB, 128), jnp.float32),
        scratch_shapes=[
            pltpu.VMEM((_NB, 128), jnp.float32),
            pltpu.VMEM((_NB, 128), jnp.float32),
        ],
    )(bx1, by1, bx2, by2)


def kernel(base_feat, img_info, gt_boxes, W1, b1, W_cls, b_cls, W_bbox, b_bbox):
    x = base_feat[0].transpose(1, 2, 0)
    x = jnp.pad(x, ((1, 1), (0, _WP - _W), (0, 0)))
    x_pad = x.reshape((_H + 2) * _WP, 256)
    w1 = W1.transpose(2, 3, 1, 0).reshape(9, 256, 512)
    wcls = W_cls[:, :, 0, 0]
    wbox = W_bbox[:, :, 0, 0]
    perm = jnp.array([a * 4 + c for c in range(4) for a in range(_A)])
    w2 = jnp.concatenate(
        [wcls, wbox[perm], jnp.zeros((10, 512), jnp.float32)], axis=0).T
    b2 = jnp.concatenate(
        [b_cls, b_bbox[perm], jnp.zeros((10,), jnp.float32)])[None, :]

    outp = _conv_decode(x_pad, w1, b1[None, :], w2, b2,
                        jnp.asarray(_AW), jnp.asarray(_AH),
                        jnp.asarray(_ACX), jnp.asarray(_ACY))
    out = outp.reshape(_H, _WP, 48)[:, :_W, :].reshape(_NPOS, 48)

    scores = out[:, 0:9].reshape(-1)
    x1 = out[:, 9:18].reshape(-1)
    y1 = out[:, 18:27].reshape(-1)
    x2 = out[:, 27:36].reshape(-1)
    y2 = out[:, 36:45].reshape(-1)

    top_scores, top_idx = lax.top_k(scores, _PRE_NMS)
    tx1 = x1[top_idx]
    ty1 = y1[top_idx]
    tx2 = x2[top_idx]
    ty2 = y2[top_idx]

    def to_blocks(v, fill):
        return jnp.concatenate(
            [v, jnp.full((_NMS_PAD - _PRE_NMS,), fill, jnp.float32)]
        ).reshape(_NB, 128)

    supp = _nms(to_blocks(tx1, 0.0), to_blocks(ty1, 0.0),
                to_blocks(tx2, -1.0), to_blocks(ty2, -1.0))
    suppressed = supp.reshape(-1)[:_PRE_NMS] > 0.5

    masked = jnp.where(suppressed, -1e9, top_scores)
    _, keep = lax.top_k(masked, _POST_NMS)
    final = jnp.stack(
        [jnp.zeros((_POST_NMS,), jnp.float32),
         tx1[keep], ty1[keep], tx2[keep], ty2[keep]], axis=1)
    return final, jnp.zeros(()), jnp.zeros(())
